# baseline (speedup 1.0000x reference)
import functools

import jax
import jax.numpy as jnp
from jax import lax
from jax.experimental import pallas as pl
from jax.experimental.pallas import tpu as pltpu
from jax.experimental.pallas import tpu_sc as plsc

N = 10000
D = 128
E = 320000

NC = 2
NS = 16
NW = NC * NS
CHUNK = 64
CH = 160
EP = NW * CH * CHUNK
NACC = 10080
PT = NACC // NS


@functools.lru_cache(maxsize=None)
def _make_sc_agg(width):
    mesh = plsc.VectorSubcoreMesh(core_axis_name="c", subcore_axis_name="s")

    @functools.partial(
        pl.kernel,
        out_type=jax.ShapeDtypeStruct((NC, NACC, width), jnp.float32),
        mesh=mesh,
        scratch_types=[
            pltpu.VMEM_SHARED((NACC, width), jnp.float32),
            pltpu.VMEM((CH, CHUNK), jnp.int32),
            pltpu.VMEM((CH, CHUNK), jnp.int32),
            pltpu.VMEM((CHUNK, width), jnp.float32),
            pltpu.VMEM((CHUNK, width), jnp.float32),
            pltpu.SemaphoreType.DMA,
            pltpu.SemaphoreType.DMA,
            pltpu.SemaphoreType.DMA,
            pltpu.SemaphoreType.DMA,
        ],
        compiler_params=pltpu.CompilerParams(use_tc_tiling_on_sc=False),
    )
    def sc_agg(table, srcr, dstr, zrows, parts, acc, sidx, didx, buf0, buf1,
               gsem0, gsem1, ssem0, ssem1):
        c = lax.axis_index("c")
        s = lax.axis_index("s")
        wid = s * NC + c
        pltpu.sync_copy(zrows, acc.at[pl.ds(s * PT, PT)])
        pltpu.sync_copy(srcr.at[wid], sidx)
        pltpu.sync_copy(dstr.at[wid], didx)
        plsc.subcore_barrier()

        pltpu.async_copy(table.at[sidx.at[0]], buf0, gsem0)
        pltpu.async_copy(table.at[sidx.at[1]], buf1, gsem1)

        @pl.loop(0, CH - 2, step=2)
        def _(j):
            pltpu.make_async_copy(table.at[sidx.at[j]], buf0, gsem0).wait()
            pltpu.async_copy(buf0, acc.at[didx.at[j]], ssem0, add=True)
            pltpu.make_async_copy(table.at[sidx.at[j + 1]], buf1, gsem1).wait()
            pltpu.async_copy(buf1, acc.at[didx.at[j + 1]], ssem1, add=True)
            pltpu.make_async_copy(buf0, acc.at[didx.at[j]], ssem0).wait()
            pltpu.async_copy(table.at[sidx.at[j + 2]], buf0, gsem0)
            pltpu.make_async_copy(buf1, acc.at[didx.at[j + 1]], ssem1).wait()
            pltpu.async_copy(table.at[sidx.at[j + 3]], buf1, gsem1)

        pltpu.make_async_copy(table.at[sidx.at[CH - 2]], buf0, gsem0).wait()
        pltpu.sync_copy(buf0, acc.at[didx.at[CH - 2]], add=True)
        pltpu.make_async_copy(table.at[sidx.at[CH - 1]], buf1, gsem1).wait()
        pltpu.sync_copy(buf1, acc.at[didx.at[CH - 1]], add=True)

        plsc.subcore_barrier()
        pltpu.sync_copy(acc.at[pl.ds(s * PT, PT)],
                        parts.at[c].at[pl.ds(s * PT, PT)])

    return sc_agg


def _xside_body(x, w, b, out_ref):
    out_ref[...] = lax.dot_general(
        x[...], w[...], (((1,), (1,)), ((), ())),
        preferred_element_type=jnp.float32,
        precision=lax.Precision.HIGHEST) + b[...]


def _combine0_body(p0, p1, wla, sel, zr, h_ref, rdeg_ref):
    ps = p0[...] + p1[...]
    dn = (((1,), (0,)), ((), ()))
    zl = lax.dot_general(ps, wla[...], dn,
                         preferred_element_type=jnp.float32,
                         precision=lax.Precision.HIGHEST)
    degb = lax.dot_general(ps, sel[...], dn,
                           preferred_element_type=jnp.float32,
                           precision=lax.Precision.HIGHEST)
    rdeg = 1.0 / jnp.maximum(degb, 1.0)
    z = zl * rdeg + zr[...]
    sig = 1.0 / (1.0 + jnp.exp(-z))
    h_ref[...] = z * (0.5 + 0.5 * sig)
    rdeg_ref[...] = rdeg


def _combine1_body(p0, p1, rdeg, wl, zr, out_ref):
    ps = p0[...] + p1[...]
    zl = lax.dot_general(ps, wl[...], (((1,), (1,)), ((), ())),
                         preferred_element_type=jnp.float32,
                         precision=lax.Precision.HIGHEST)
    out_ref[...] = zl * rdeg[...] + zr[...]


_RB = 2000
_GRID = N // _RB


def _row_spec(width):
    return pl.BlockSpec((_RB, width), lambda i: (i, 0))


def _full_spec(shape):
    return pl.BlockSpec(shape, lambda i: tuple(0 for _ in shape))


def _xside(x, w, b):
    return pl.pallas_call(
        _xside_body,
        grid=(_GRID,),
        in_specs=[_row_spec(D), _full_spec((D, D)), _full_spec((1, D))],
        out_specs=_row_spec(D),
        out_shape=jax.ShapeDtypeStruct((N, D), jnp.float32),
    )(x, w, b)


def _combine0(p0, p1, wla, sel, zr):
    return pl.pallas_call(
        _combine0_body,
        grid=(_GRID,),
        in_specs=[
            _row_spec(D + 16), _row_spec(D + 16),
            _full_spec((D + 16, D)), _full_spec((D + 16, D)),
            _row_spec(D),
        ],
        out_specs=[_row_spec(D), _row_spec(D)],
        out_shape=[jax.ShapeDtypeStruct((N, D), jnp.float32),
                   jax.ShapeDtypeStruct((N, D), jnp.float32)],
    )(p0, p1, wla, sel, zr)


def _combine1(p0, p1, rdeg, wl, zr):
    return pl.pallas_call(
        _combine1_body,
        grid=(_GRID,),
        in_specs=[
            _row_spec(D), _row_spec(D), _row_spec(D),
            _full_spec((D, D)), _row_spec(D),
        ],
        out_specs=_row_spec(D),
        out_shape=jax.ShapeDtypeStruct((N, D), jnp.float32),
    )(p0, p1, rdeg, wl, zr)


def kernel(x, edge_index, W_l0, b_l0, W_r0, W_l1, b_l1, W_r1):
    src = edge_index[0]
    dst = edge_index[1]
    pad = EP - E
    pad_iota = jnp.arange(pad, dtype=jnp.int32)
    srcp = jnp.concatenate([src, pad_iota % N]).reshape(NW, CH, CHUNK)
    dstp = jnp.concatenate([dst, N + pad_iota % (NACC - N)]).reshape(
        NW, CH, CHUNK)

    x_aug = jnp.concatenate([x, jnp.ones((N, 16), jnp.float32)], axis=1)
    parts0 = _make_sc_agg(D + 16)(x_aug, srcp, dstp,
                                  jnp.zeros((PT, D + 16), jnp.float32))
    zr0 = _xside(x, W_r0, b_l0.reshape(1, D))

    wla = jnp.concatenate([W_l0.T, jnp.zeros((16, D), jnp.float32)], axis=0)
    sel = jnp.zeros((D + 16, D), jnp.float32).at[D, :].set(1.0)
    h0, rdeg = _combine0(parts0[0], parts0[1], wla, sel, zr0)

    parts1 = _make_sc_agg(D)(h0, srcp, dstp, jnp.zeros((PT, D), jnp.float32))
    zr1 = _xside(h0, W_r1, b_l1.reshape(1, D))

    return _combine1(parts1[0], parts1[1], rdeg, W_l1, zr1)


# speedup vs baseline: 1.1873x; 1.1873x over previous
import functools

import jax
import jax.numpy as jnp
from jax import lax
from jax.experimental import pallas as pl
from jax.experimental.pallas import tpu as pltpu
from jax.experimental.pallas import tpu_sc as plsc

N = 10000
D = 128
E = 320000

NC = 2
NS = 16
NW = NC * NS
CHUNK = 64
CH = 160
EP = NW * CH * CHUNK
NACC = 10080
PT = NACC // NS


@functools.lru_cache(maxsize=None)
def _make_sc_agg(width):
    mesh = plsc.VectorSubcoreMesh(core_axis_name="c", subcore_axis_name="s")

    @functools.partial(
        pl.kernel,
        out_type=jax.ShapeDtypeStruct((NC, NACC, width), jnp.float32),
        mesh=mesh,
        scratch_types=[
            pltpu.VMEM_SHARED((NACC, width), jnp.float32),
            pltpu.VMEM((CH, CHUNK), jnp.int32),
            pltpu.VMEM((CH, CHUNK), jnp.int32),
            pltpu.VMEM((CHUNK, width), jnp.float32),
            pltpu.VMEM((CHUNK, width), jnp.float32),
            pltpu.SemaphoreType.DMA,
            pltpu.SemaphoreType.DMA,
            pltpu.SemaphoreType.DMA,
            pltpu.SemaphoreType.DMA,
        ],
        compiler_params=pltpu.CompilerParams(use_tc_tiling_on_sc=False),
    )
    def sc_agg(table, srcr, dstr, zrows, parts, acc, sidx, didx, buf0, buf1,
               gsem0, gsem1, ssem0, ssem1):
        c = lax.axis_index("c")
        s = lax.axis_index("s")
        wid = s * NC + c
        pltpu.sync_copy(zrows, acc.at[pl.ds(s * PT, PT)])
        pltpu.sync_copy(srcr.at[wid], sidx)
        pltpu.sync_copy(dstr.at[wid], didx)
        plsc.subcore_barrier()

        pltpu.async_copy(table.at[sidx.at[0]], buf0, gsem0)
        pltpu.async_copy(table.at[sidx.at[1]], buf1, gsem1)

        @pl.loop(0, CH - 2, step=2)
        def _(j):
            pltpu.make_async_copy(table.at[sidx.at[j]], buf0, gsem0).wait()
            pltpu.sync_copy(buf0, acc.at[didx.at[j]], add=True)
            pltpu.async_copy(table.at[sidx.at[j + 2]], buf0, gsem0)
            pltpu.make_async_copy(table.at[sidx.at[j + 1]], buf1, gsem1).wait()
            pltpu.sync_copy(buf1, acc.at[didx.at[j + 1]], add=True)
            pltpu.async_copy(table.at[sidx.at[j + 3]], buf1, gsem1)

        pltpu.make_async_copy(table.at[sidx.at[CH - 2]], buf0, gsem0).wait()
        pltpu.sync_copy(buf0, acc.at[didx.at[CH - 2]], add=True)
        pltpu.make_async_copy(table.at[sidx.at[CH - 1]], buf1, gsem1).wait()
        pltpu.sync_copy(buf1, acc.at[didx.at[CH - 1]], add=True)

        plsc.subcore_barrier()
        pltpu.sync_copy(acc.at[pl.ds(s * PT, PT)],
                        parts.at[c].at[pl.ds(s * PT, PT)])

    return sc_agg


def _xside_body(x, w, b, out_ref):
    out_ref[...] = lax.dot_general(
        x[...], w[...], (((1,), (1,)), ((), ())),
        preferred_element_type=jnp.float32,
        precision=lax.Precision.HIGHEST) + b[...]


def _combine0_body(p0, p1, wla, sel, zr, h_ref, rdeg_ref):
    ps = p0[...] + p1[...]
    dn = (((1,), (0,)), ((), ()))
    zl = lax.dot_general(ps, wla[...], dn,
                         preferred_element_type=jnp.float32,
                         precision=lax.Precision.HIGHEST)
    degb = lax.dot_general(ps, sel[...], dn,
                           preferred_element_type=jnp.float32,
                           precision=lax.Precision.HIGHEST)
    rdeg = 1.0 / jnp.maximum(degb, 1.0)
    z = zl * rdeg + zr[...]
    sig = 1.0 / (1.0 + jnp.exp(-z))
    h_ref[...] = z * (0.5 + 0.5 * sig)
    rdeg_ref[...] = rdeg


def _combine1_body(p0, p1, rdeg, wl, zr, out_ref):
    ps = p0[...] + p1[...]
    zl = lax.dot_general(ps, wl[...], (((1,), (1,)), ((), ())),
                         preferred_element_type=jnp.float32,
                         precision=lax.Precision.HIGHEST)
    out_ref[...] = zl * rdeg[...] + zr[...]


_RB = 2000
_GRID = N // _RB


def _row_spec(width):
    return pl.BlockSpec((_RB, width), lambda i: (i, 0))


def _full_spec(shape):
    return pl.BlockSpec(shape, lambda i: tuple(0 for _ in shape))


def _xside(x, w, b):
    return pl.pallas_call(
        _xside_body,
        grid=(_GRID,),
        in_specs=[_row_spec(D), _full_spec((D, D)), _full_spec((1, D))],
        out_specs=_row_spec(D),
        out_shape=jax.ShapeDtypeStruct((N, D), jnp.float32),
    )(x, w, b)


def _combine0(p0, p1, wla, sel, zr):
    return pl.pallas_call(
        _combine0_body,
        grid=(_GRID,),
        in_specs=[
            _row_spec(D + 16), _row_spec(D + 16),
            _full_spec((D + 16, D)), _full_spec((D + 16, D)),
            _row_spec(D),
        ],
        out_specs=[_row_spec(D), _row_spec(D)],
        out_shape=[jax.ShapeDtypeStruct((N, D), jnp.float32),
                   jax.ShapeDtypeStruct((N, D), jnp.float32)],
    )(p0, p1, wla, sel, zr)


def _combine1(p0, p1, rdeg, wl, zr):
    return pl.pallas_call(
        _combine1_body,
        grid=(_GRID,),
        in_specs=[
            _row_spec(D), _row_spec(D), _row_spec(D),
            _full_spec((D, D)), _row_spec(D),
        ],
        out_specs=_row_spec(D),
        out_shape=jax.ShapeDtypeStruct((N, D), jnp.float32),
    )(p0, p1, rdeg, wl, zr)


def kernel(x, edge_index, W_l0, b_l0, W_r0, W_l1, b_l1, W_r1):
    src = edge_index[0]
    dst = edge_index[1]
    pad = EP - E
    pad_iota = jnp.arange(pad, dtype=jnp.int32)
    srcp = jnp.concatenate([src, pad_iota % N]).reshape(NW, CH, CHUNK)
    dstp = jnp.concatenate([dst, N + pad_iota % (NACC - N)]).reshape(
        NW, CH, CHUNK)

    x_aug = jnp.concatenate([x, jnp.ones((N, 16), jnp.float32)], axis=1)
    parts0 = _make_sc_agg(D + 16)(x_aug, srcp, dstp,
                                  jnp.zeros((PT, D + 16), jnp.float32))
    zr0 = _xside(x, W_r0, b_l0.reshape(1, D))

    wla = jnp.concatenate([W_l0.T, jnp.zeros((16, D), jnp.float32)], axis=0)
    sel = jnp.zeros((D + 16, D), jnp.float32).at[D, :].set(1.0)
    h0, rdeg = _combine0(parts0[0], parts0[1], wla, sel, zr0)

    parts1 = _make_sc_agg(D)(h0, srcp, dstp, jnp.zeros((PT, D), jnp.float32))
    zr1 = _xside(h0, W_r1, b_l1.reshape(1, D))

    return _combine1(parts1[0], parts1[1], rdeg, W_l1, zr1)
